# baseline (speedup 1.0000x reference)
import functools

import jax
import jax.numpy as jnp
from jax import lax
from jax.experimental import pallas as pl
from jax.experimental.pallas import tpu as pltpu
from jax.experimental.pallas import tpu_sc as plsc

NUM_IND = 128
DIM = 32
BATCH = 16384
NUM_CORES = 2
NUM_SUBCORES = 16
NW = NUM_CORES * NUM_SUBCORES
SC_ROWS = 4096
TC_ROWS = BATCH - SC_ROWS
ROWS_PER_W = SC_ROWS // NW
TC_BLK = 2048
TC_NBLK = TC_ROWS // TC_BLK


def _table_body(vars_ref, w1_ref, b1_ref, w2_ref, b2_ref, emb_ref, out_ref):
    h = lax.dot_general(
        vars_ref[...], w1_ref[...], (((1,), (0,)), ((), ())),
        preferred_element_type=jnp.float32,
        precision=lax.Precision.HIGHEST)
    h = jnp.maximum(h + b1_ref[...], 0.0)
    proj = lax.dot_general(
        h, w2_ref[...], (((1,), (0,)), ((), ())),
        preferred_element_type=jnp.float32,
        precision=lax.Precision.HIGHEST)
    out_ref[...] = proj + b2_ref[...] + 0.1 * emb_ref[...]


_table = pl.pallas_call(
    _table_body,
    out_shape=jax.ShapeDtypeStruct((NUM_IND, DIM), jnp.float32),
)


def _onehot_body(idx_ref, table_ref, sc_ref, out_ref):
    ids = idx_ref[0, :]
    onehot = (ids[:, None] == lax.broadcasted_iota(
        jnp.int32, (1, NUM_IND), 1)).astype(jnp.float32)
    out_ref[0:TC_ROWS, :] = lax.dot_general(
        onehot, table_ref[...], (((1,), (0,)), ((), ())),
        preferred_element_type=jnp.float32)
    out_ref[TC_ROWS:BATCH, :] = sc_ref[...]


_onehot_gather = pl.pallas_call(
    _onehot_body,
    out_shape=jax.ShapeDtypeStruct((BATCH, DIM), jnp.float32),
)


@functools.partial(
    pl.kernel,
    out_type=jax.ShapeDtypeStruct((SC_ROWS, DIM), jnp.float32),
    mesh=plsc.VectorSubcoreMesh(
        core_axis_name="c", subcore_axis_name="s",
        num_cores=NUM_CORES, num_subcores=NUM_SUBCORES),
    scratch_types=[
        pltpu.VMEM((ROWS_PER_W,), jnp.int32),
        pltpu.VMEM((ROWS_PER_W, DIM), jnp.float32),
        pltpu.SemaphoreType.DMA,
    ],
    compiler_params=pltpu.CompilerParams(use_tc_tiling_on_sc=False),
)
def _sc_gather(table_hbm, idx_hbm, out_hbm, idx_v, rows_v, sem):
    wid = lax.axis_index("s") * NUM_CORES + lax.axis_index("c")
    pltpu.sync_copy(idx_hbm.at[pl.ds(wid * ROWS_PER_W, ROWS_PER_W)], idx_v)
    pltpu.async_copy(table_hbm.at[idx_v], rows_v, sem).wait()
    pltpu.sync_copy(rows_v, out_hbm.at[pl.ds(wid * ROWS_PER_W, ROWS_PER_W)])


def kernel(industry_vars, W1, b1, W2, b2, emb, industry_idx):
    table = _table(industry_vars, W1, b1.reshape(1, -1), W2,
                   b2.reshape(1, -1), emb)
    idx = industry_idx.astype(jnp.int32)
    sc_out = _sc_gather(table, idx[TC_ROWS:])
    return _onehot_gather(idx[:TC_ROWS].reshape(1, TC_ROWS), table, sc_out)


# speedup vs baseline: 1.1156x; 1.1156x over previous
import functools

import jax
import jax.numpy as jnp
from jax import lax
from jax.experimental import pallas as pl
from jax.experimental.pallas import tpu as pltpu
from jax.experimental.pallas import tpu_sc as plsc

NUM_IND = 128
DIM = 32
BATCH = 16384
NUM_CORES = 2
NUM_SUBCORES = 16
NW = NUM_CORES * NUM_SUBCORES
SC_ROWS = 2048
TC_ROWS = BATCH - SC_ROWS
ROWS_PER_W = SC_ROWS // NW
TC_BLK = 2048
TC_NBLK = TC_ROWS // TC_BLK


def _table_body(vars_ref, w1_ref, b1_ref, w2_ref, b2_ref, emb_ref, out_ref):
    h = lax.dot_general(
        vars_ref[...], w1_ref[...], (((1,), (0,)), ((), ())),
        preferred_element_type=jnp.float32,
        precision=lax.Precision.HIGHEST)
    h = jnp.maximum(h + b1_ref[...], 0.0)
    proj = lax.dot_general(
        h, w2_ref[...], (((1,), (0,)), ((), ())),
        preferred_element_type=jnp.float32,
        precision=lax.Precision.HIGHEST)
    out_ref[...] = proj + b2_ref[...] + 0.1 * emb_ref[...]


_table = pl.pallas_call(
    _table_body,
    out_shape=jax.ShapeDtypeStruct((NUM_IND, DIM), jnp.float32),
)


def _onehot_body(idx_ref, table_ref, out_ref):
    ids = idx_ref[0, :]
    onehot = (ids[:, None] == lax.broadcasted_iota(
        jnp.int32, (1, NUM_IND), 1)).astype(jnp.float32)
    out_ref[...] = lax.dot_general(
        onehot, table_ref[...], (((1,), (0,)), ((), ())),
        preferred_element_type=jnp.float32)


_onehot_gather = pl.pallas_call(
    _onehot_body,
    out_shape=jax.ShapeDtypeStruct((TC_ROWS, DIM), jnp.float32),
)


@functools.partial(
    pl.kernel,
    out_type=jax.ShapeDtypeStruct((SC_ROWS, DIM), jnp.float32),
    mesh=plsc.VectorSubcoreMesh(
        core_axis_name="c", subcore_axis_name="s",
        num_cores=NUM_CORES, num_subcores=NUM_SUBCORES),
    scratch_types=[
        pltpu.VMEM((ROWS_PER_W,), jnp.int32),
        pltpu.VMEM((ROWS_PER_W, DIM), jnp.float32),
        pltpu.SemaphoreType.DMA,
    ],
    compiler_params=pltpu.CompilerParams(use_tc_tiling_on_sc=False),
)
def _sc_gather(table_hbm, idx_hbm, out_hbm, idx_v, rows_v, sem):
    wid = lax.axis_index("s") * NUM_CORES + lax.axis_index("c")
    pltpu.sync_copy(idx_hbm.at[pl.ds(wid * ROWS_PER_W, ROWS_PER_W)], idx_v)
    pltpu.async_copy(table_hbm.at[idx_v], rows_v, sem).wait()
    pltpu.sync_copy(rows_v, out_hbm.at[pl.ds(wid * ROWS_PER_W, ROWS_PER_W)])


def kernel(industry_vars, W1, b1, W2, b2, emb, industry_idx):
    table = _table(industry_vars, W1, b1.reshape(1, -1), W2,
                   b2.reshape(1, -1), emb)
    idx = industry_idx.astype(jnp.int32)
    sc_out = _sc_gather(table, idx[TC_ROWS:])
    tc_out = _onehot_gather(idx[:TC_ROWS].reshape(1, TC_ROWS), table)
    return jnp.concatenate([tc_out, sc_out], axis=0)


# speedup vs baseline: 1.2188x; 1.0925x over previous
import functools

import jax
import jax.numpy as jnp
from jax import lax
from jax.experimental import pallas as pl
from jax.experimental.pallas import tpu as pltpu
from jax.experimental.pallas import tpu_sc as plsc

NUM_IND = 128
DIM = 32
BATCH = 16384
NUM_CORES = 2
NUM_SUBCORES = 16
NW = NUM_CORES * NUM_SUBCORES
SC_ROWS = 4096
TC_ROWS = BATCH - SC_ROWS
ROWS_PER_W = SC_ROWS // NW
TC_BLK = 2048
TC_NBLK = TC_ROWS // TC_BLK


def _table_body(vars_ref, w1_ref, b1_ref, w2_ref, b2_ref, emb_ref, out_ref):
    h = lax.dot_general(
        vars_ref[...], w1_ref[...], (((1,), (0,)), ((), ())),
        preferred_element_type=jnp.float32,
        precision=lax.Precision.HIGHEST)
    h = jnp.maximum(h + b1_ref[...], 0.0)
    proj = lax.dot_general(
        h, w2_ref[...], (((1,), (0,)), ((), ())),
        preferred_element_type=jnp.float32,
        precision=lax.Precision.HIGHEST)
    out_ref[...] = proj + b2_ref[...] + 0.1 * emb_ref[...]


_table = pl.pallas_call(
    _table_body,
    out_shape=jax.ShapeDtypeStruct((NUM_IND, DIM), jnp.float32),
)


def _onehot_body(idx_ref, table_ref, out_ref):
    ids = idx_ref[0, 0:TC_ROWS]
    onehot = (ids[:, None] == lax.broadcasted_iota(
        jnp.int32, (1, NUM_IND), 1)).astype(jnp.float32)
    out_ref[...] = lax.dot_general(
        onehot, table_ref[...], (((1,), (0,)), ((), ())),
        preferred_element_type=jnp.float32)


_onehot_gather = pl.pallas_call(
    _onehot_body,
    out_shape=jax.ShapeDtypeStruct((TC_ROWS, DIM), jnp.float32),
)


@functools.partial(
    pl.kernel,
    out_type=jax.ShapeDtypeStruct((SC_ROWS, DIM), jnp.float32),
    mesh=plsc.VectorSubcoreMesh(
        core_axis_name="c", subcore_axis_name="s",
        num_cores=NUM_CORES, num_subcores=NUM_SUBCORES),
    scratch_types=[
        pltpu.VMEM((ROWS_PER_W,), jnp.int32),
        pltpu.VMEM((ROWS_PER_W, DIM), jnp.float32),
        pltpu.SemaphoreType.DMA,
    ],
    compiler_params=pltpu.CompilerParams(use_tc_tiling_on_sc=False),
)
def _sc_gather(table_hbm, idx_hbm, out_hbm, idx_v, rows_v, sem):
    wid = lax.axis_index("s") * NUM_CORES + lax.axis_index("c")
    pltpu.sync_copy(
        idx_hbm.at[pl.ds(TC_ROWS + wid * ROWS_PER_W, ROWS_PER_W)], idx_v)
    pltpu.async_copy(table_hbm.at[idx_v], rows_v, sem).wait()
    pltpu.sync_copy(rows_v, out_hbm.at[pl.ds(wid * ROWS_PER_W, ROWS_PER_W)])


def kernel(industry_vars, W1, b1, W2, b2, emb, industry_idx):
    table = _table(industry_vars, W1, b1.reshape(1, -1), W2,
                   b2.reshape(1, -1), emb)
    idx = industry_idx.astype(jnp.int32)
    sc_out = _sc_gather(table, idx)
    tc_out = _onehot_gather(idx.reshape(1, BATCH), table)
    return jnp.concatenate([tc_out, sc_out], axis=0)
